# baseline (speedup 1.0000x reference)
import functools

import jax
import jax.numpy as jnp
from jax import lax
from jax.experimental import pallas as pl
from jax.experimental.pallas import tpu as pltpu
from jax.experimental.pallas import tpu_sc as plsc

_HIDDEN = 64
_SCALE = 8.0


@functools.cache
def _make_lookup(B):
    info = plsc.get_sparse_core_info()
    NC, NS, L = info.num_cores, info.num_subcores, info.num_lanes
    NW = NC * NS
    assert B % NW == 0
    b_per_w = B // NW
    C = 1600
    n_chunks = b_per_w // C
    assert b_per_w % C == 0

    mesh = plsc.VectorSubcoreMesh(core_axis_name="c", subcore_axis_name="s")

    @functools.partial(
        pl.kernel,
        out_type=jax.ShapeDtypeStruct((B, _HIDDEN), jnp.float32),
        mesh=mesh,
        scratch_types=[
            pltpu.VMEM((C,), jnp.int32),
            pltpu.VMEM((C, _HIDDEN), jnp.float32),
            pltpu.SemaphoreType.DMA,
        ],
        compiler_params=pltpu.CompilerParams(use_tc_tiling_on_sc=False),
    )
    def lookup(idx_hbm, table_hbm, out_hbm, idx_v, rows_v, sem):
        wid = lax.axis_index("s") * NC + lax.axis_index("c")
        base = wid * b_per_w

        def chunk_body(c, carry):
            start = base + c * C
            pltpu.sync_copy(idx_hbm.at[pl.ds(start, C)], idx_v)
            pltpu.async_copy(table_hbm.at[idx_v], rows_v, sem).wait()

            def scale_row(i, carry2):
                for j in range(_HIDDEN // L):
                    rows_v[i, pl.ds(j * L, L)] = (
                        rows_v[i, pl.ds(j * L, L)] * _SCALE
                    )
                return carry2

            lax.fori_loop(0, C, scale_row, 0)
            pltpu.sync_copy(rows_v, out_hbm.at[pl.ds(start, C)])
            return carry

        lax.fori_loop(0, n_chunks, chunk_body, 0)

    return lookup


def kernel(x, table):
    B = x.shape[0] * x.shape[1]
    flat = x.reshape(B).astype(jnp.int32)
    out = _make_lookup(B)(flat, table)
    return out.reshape(x.shape[0], x.shape[1], _HIDDEN)


# speedup vs baseline: 1.0909x; 1.0909x over previous
import functools

import jax
import jax.numpy as jnp
from jax import lax
from jax.experimental import pallas as pl
from jax.experimental.pallas import tpu as pltpu
from jax.experimental.pallas import tpu_sc as plsc

_HIDDEN = 64
_SCALE = 8.0


@functools.cache
def _make_lookup(B):
    info = plsc.get_sparse_core_info()
    NC, NS, L = info.num_cores, info.num_subcores, info.num_lanes
    NW = NC * NS
    assert B % NW == 0
    b_per_w = B // NW
    C = 800
    n_chunks = b_per_w // C
    assert b_per_w % C == 0

    mesh = plsc.VectorSubcoreMesh(core_axis_name="c", subcore_axis_name="s")

    @functools.partial(
        pl.kernel,
        out_type=jax.ShapeDtypeStruct((B, _HIDDEN), jnp.float32),
        mesh=mesh,
        scratch_types=[
            pltpu.VMEM((C,), jnp.int32),
            pltpu.VMEM((C,), jnp.int32),
            pltpu.VMEM((C, _HIDDEN), jnp.float32),
            pltpu.VMEM((C, _HIDDEN), jnp.float32),
            pltpu.SemaphoreType.DMA,
            pltpu.SemaphoreType.DMA,
            pltpu.SemaphoreType.DMA,
            pltpu.SemaphoreType.DMA,
        ],
        compiler_params=pltpu.CompilerParams(use_tc_tiling_on_sc=False),
    )
    def lookup(idx_hbm, table_hbm, out_hbm, i0, i1, r0, r1, g0, g1, s0, s1):
        idx_v = (i0, i1)
        rows_v = (r0, r1)
        gsem = (g0, g1)
        ssem = (s0, s1)
        wid = lax.axis_index("s") * NC + lax.axis_index("c")
        base = wid * b_per_w

        def start_gather(c, b):
            start = base + c * C
            pltpu.sync_copy(idx_hbm.at[pl.ds(start, C)], idx_v[b])
            return pltpu.async_copy(table_hbm.at[idx_v[b]], rows_v[b], gsem[b])

        def scale(ref):
            @plsc.parallel_loop(0, C, step=1, unroll=8)
            def _(i):
                for j in range(_HIDDEN // L):
                    ref[i, pl.ds(j * L, L)] = ref[i, pl.ds(j * L, L)] * _SCALE

        gathers = {0: start_gather(0, 0)}
        scatters = {}
        for c in range(n_chunks):
            b = c % 2
            gathers[c].wait()
            if c + 1 < n_chunks:
                if c - 1 >= 0:
                    scatters[c - 1].wait()
                gathers[c + 1] = start_gather(c + 1, 1 - b)
            scale(rows_v[b])
            scatters[c] = pltpu.async_copy(
                rows_v[b], out_hbm.at[pl.ds(base + c * C, C)], ssem[b]
            )
        if n_chunks >= 2:
            scatters[n_chunks - 2].wait()
        scatters[n_chunks - 1].wait()

    return lookup


def kernel(x, table):
    B = x.shape[0] * x.shape[1]
    flat = x.reshape(B).astype(jnp.int32)
    out = _make_lookup(B)(flat, table)
    return out.reshape(x.shape[0], x.shape[1], _HIDDEN)
